# baseline (speedup 1.0000x reference)
import functools

import jax
import jax.numpy as jnp
from jax import lax
from jax.experimental import pallas as pl
from jax.experimental.pallas import tpu as pltpu
from jax.experimental.pallas import tpu_sc as plsc

L1_CAP, L2_CAP, L3_CAP = 2048, 4096, 8192
SDR, CDIM = 2048, 384
N = 2048

_BLK = 512
_SC_NC = 2
_SC_NS = 16
_NW = _SC_NC * _SC_NS
_PER = 64


_S1 = N // _BLK
_G1 = _S1
_G2 = _G1 + 2 * _S1
_G3 = _G2 + (L3_CAP // _BLK)


def _sdr_body(sdrs, l1s, l2s, l3s, o1, o2, o3):
    i = pl.program_id(0)

    @pl.when(i < _G1)
    def _():
        o1[...] = sdrs[...]

    @pl.when(jnp.logical_and(i >= _G1, i < _G1 + _S1))
    def _():
        o2[...] = l1s[...]

    @pl.when(jnp.logical_and(i >= _G1 + _S1, i < _G2))
    def _():
        o2[...] = l2s[...]

    @pl.when(jnp.logical_and(i >= _G2, i < _G2 + _S1))
    def _():
        o3[...] = l2s[...]

    @pl.when(i >= _G2 + _S1)
    def _():
        o3[...] = l3s[...]


def _tc_sdr_copy(sdrs, l1s, l2s, l3s):
    def sdrs_map(i):
        return (jnp.minimum(i, _G1 - 1), 0)

    def l1s_map(i):
        return (jnp.clip(i - _G1, 0, _S1 - 1), 0)

    def l2s_map(i):
        j = jnp.clip(i, _G1 + _S1, _G2 + _S1 - 1)
        return (jnp.where(j < _G2, j - _G1, j - _G2), 0)

    def l3s_map(i):
        return (jnp.clip(i - _G2, _S1, L3_CAP // _BLK - 1), 0)

    def o1_map(i):
        return (jnp.minimum(i, _G1 - 1), 0)

    def o2_map(i):
        return (jnp.clip(i - _G1, 0, 2 * _S1 - 1), 0)

    def o3_map(i):
        return (jnp.clip(i - _G2, 0, L3_CAP // _BLK - 1), 0)

    blk = (_BLK, SDR)
    return pl.pallas_call(
        _sdr_body,
        grid=(_G3,),
        in_specs=[pl.BlockSpec(blk, m)
                  for m in (sdrs_map, l1s_map, l2s_map, l3s_map)],
        out_specs=[pl.BlockSpec(blk, m) for m in (o1_map, o2_map, o3_map)],
        out_shape=[
            jax.ShapeDtypeStruct((L1_CAP, SDR), jnp.float32),
            jax.ShapeDtypeStruct((L2_CAP, SDR), jnp.float32),
            jax.ShapeDtypeStruct((L3_CAP, SDR), jnp.float32),
        ],
    )(sdrs, l1s, l2s, l3s)


def _sc_body(contents, l1c, l2c, l3c, o1c, o2c, o3c, b0, b1, sin, sout):
    w = lax.axis_index("s") * _SC_NC + lax.axis_index("c")
    off = w * _PER
    regs = [
        (contents, 0, o1c, 0),
        (l1c, 0, o2c, 0),
        (l2c, N, o2c, N),
        (l2c, 0, o3c, 0),
        (l3c, N, o3c, N),
        (l3c, N + _NW * _PER, o3c, N + _NW * _PER),
        (l3c, N + 2 * _NW * _PER, o3c, N + 2 * _NW * _PER),
    ]
    bufs = (b0, b1)
    n = len(regs)
    cins, couts = [], []
    for i, (src, s0, dst, d0) in enumerate(regs):
        buf = bufs[i % 2]
        cins.append(pltpu.make_async_copy(
            src.at[pl.ds(s0 + off, _PER)], buf, sin.at[i % 2]))
        couts.append(pltpu.make_async_copy(
            buf, dst.at[pl.ds(d0 + off, _PER)], sout.at[i % 2]))
    cins[0].start()
    for i in range(n):
        cins[i].wait()
        couts[i].start()
        if i + 1 < n:
            if i >= 1:
                couts[i - 1].wait()
            cins[i + 1].start()
    couts[n - 2].wait()
    couts[n - 1].wait()


def _sc_content_copy(contents, l1c, l2c, l3c):
    mesh = plsc.VectorSubcoreMesh(
        core_axis_name="c", subcore_axis_name="s",
        num_cores=_SC_NC, num_subcores=_SC_NS)
    f = pl.kernel(
        _sc_body,
        out_type=[
            jax.ShapeDtypeStruct((L1_CAP, CDIM), jnp.float32),
            jax.ShapeDtypeStruct((L2_CAP, CDIM), jnp.float32),
            jax.ShapeDtypeStruct((L3_CAP, CDIM), jnp.float32),
        ],
        mesh=mesh,
        scratch_types=[
            pltpu.VMEM((_PER, CDIM), jnp.float32),
            pltpu.VMEM((_PER, CDIM), jnp.float32),
            pltpu.SemaphoreType.DMA((2,)),
            pltpu.SemaphoreType.DMA((2,)),
        ],
    )
    return f(contents, l1c, l2c, l3c)


def kernel(sdrs, contents, l1_sdr_bank, l1_content_bank,
           l2_sdr_bank, l2_content_bank, l3_sdr_bank, l3_content_bank):
    sdrs = jax.lax.stop_gradient(sdrs)
    contents = jax.lax.stop_gradient(contents)

    o1c, o2c, o3c = _sc_content_copy(
        contents, l1_content_bank, l2_content_bank, l3_content_bank)

    o1s, o2s, o3s = _tc_sdr_copy(sdrs, l1_sdr_bank, l2_sdr_bank, l3_sdr_bank)
    return (o1s, o1c, o2s, o2c, o3s, o3c)


# speedup vs baseline: 1.2198x; 1.2198x over previous
import functools

import jax
import jax.numpy as jnp
from jax.experimental import pallas as pl
from jax.experimental.pallas import tpu as pltpu

L1_CAP, L2_CAP, L3_CAP = 2048, 4096, 8192
SDR, CDIM = 2048, 384
N = 2048

_BLK = 1024


def _copy2_body(a_s, a_c, o_s, o_c):
    o_s[...] = a_s[...]
    o_c[...] = a_c[...]


def _concat_body(split, a_s, a_c, b_s, b_c, o_s, o_c):
    i = pl.program_id(0)

    @pl.when(i < split)
    def _():
        o_s[...] = a_s[...]
        o_c[...] = a_c[...]

    @pl.when(i >= split)
    def _():
        o_s[...] = b_s[...]
        o_c[...] = b_c[...]


def _tier_copy(a_s, a_c):
    rows = a_s.shape[0]
    grid = rows // _BLK
    return pl.pallas_call(
        _copy2_body,
        grid=(grid,),
        in_specs=[
            pl.BlockSpec((_BLK, SDR), lambda i: (i, 0)),
            pl.BlockSpec((_BLK, CDIM), lambda i: (i, 0)),
        ],
        out_specs=[
            pl.BlockSpec((_BLK, SDR), lambda i: (i, 0)),
            pl.BlockSpec((_BLK, CDIM), lambda i: (i, 0)),
        ],
        out_shape=[
            jax.ShapeDtypeStruct((rows, SDR), jnp.float32),
            jax.ShapeDtypeStruct((rows, CDIM), jnp.float32),
        ],
    )(a_s, a_c)


def _tier_concat(a_s, a_c, b_s, b_c, rows, a_rows, b_row0):
    grid = rows // _BLK
    split = a_rows // _BLK
    boff = b_row0 // _BLK

    def a_map(i):
        return (jnp.minimum(i, split - 1), 0)

    def b_map(i):
        return (jnp.maximum(i, split) - split + boff, 0)

    return pl.pallas_call(
        functools.partial(_concat_body, split),
        grid=(grid,),
        in_specs=[
            pl.BlockSpec((_BLK, SDR), a_map),
            pl.BlockSpec((_BLK, CDIM), a_map),
            pl.BlockSpec((_BLK, SDR), b_map),
            pl.BlockSpec((_BLK, CDIM), b_map),
        ],
        out_specs=[
            pl.BlockSpec((_BLK, SDR), lambda i: (i, 0)),
            pl.BlockSpec((_BLK, CDIM), lambda i: (i, 0)),
        ],
        out_shape=[
            jax.ShapeDtypeStruct((rows, SDR), jnp.float32),
            jax.ShapeDtypeStruct((rows, CDIM), jnp.float32),
        ],
    )(a_s, a_c, b_s, b_c)


def kernel(sdrs, contents, l1_sdr_bank, l1_content_bank,
           l2_sdr_bank, l2_content_bank, l3_sdr_bank, l3_content_bank):
    sdrs = jax.lax.stop_gradient(sdrs)
    contents = jax.lax.stop_gradient(contents)

    o1s, o1c = _tier_copy(sdrs, contents)
    o2s, o2c = _tier_concat(l1_sdr_bank, l1_content_bank,
                            l2_sdr_bank, l2_content_bank,
                            rows=L2_CAP, a_rows=N, b_row0=N)
    o3s, o3c = _tier_concat(l2_sdr_bank, l2_content_bank,
                            l3_sdr_bank, l3_content_bank,
                            rows=L3_CAP, a_rows=N, b_row0=N)
    return (o1s, o1c, o2s, o2c, o3s, o3c)


# speedup vs baseline: 1.2211x; 1.0011x over previous
import functools

import jax
import jax.numpy as jnp
from jax.experimental import pallas as pl

L1_CAP, L2_CAP, L3_CAP = 2048, 4096, 8192
SDR, CDIM = 2048, 384
N = 2048

_BLK = 1024


def _copy2_body(a_s, a_c, o_s, o_c):
    o_s[...] = a_s[...]
    o_c[...] = a_c[...]


def _concat_body(split, a_s, a_c, b_s, b_c, o_s, o_c):
    i = pl.program_id(0)

    @pl.when(i < split)
    def _():
        o_s[...] = a_s[...]
        o_c[...] = a_c[...]

    @pl.when(i >= split)
    def _():
        o_s[...] = b_s[...]
        o_c[...] = b_c[...]


def _tier_copy(a_s, a_c):
    rows = a_s.shape[0]
    grid = rows // _BLK
    return pl.pallas_call(
        _copy2_body,
        grid=(grid,),
        in_specs=[
            pl.BlockSpec((_BLK, SDR), lambda i: (i, 0)),
            pl.BlockSpec((_BLK, CDIM), lambda i: (i, 0)),
        ],
        out_specs=[
            pl.BlockSpec((_BLK, SDR), lambda i: (i, 0)),
            pl.BlockSpec((_BLK, CDIM), lambda i: (i, 0)),
        ],
        out_shape=[
            jax.ShapeDtypeStruct((rows, SDR), jnp.float32),
            jax.ShapeDtypeStruct((rows, CDIM), jnp.float32),
        ],
    )(a_s, a_c)


def _tier_concat(a_s, a_c, b_s, b_c, rows, a_rows, b_row0):
    grid = rows // _BLK
    split = a_rows // _BLK
    boff = b_row0 // _BLK

    def a_map(i):
        return (jnp.minimum(i, split - 1), 0)

    def b_map(i):
        return (jnp.maximum(i, split) - split + boff, 0)

    return pl.pallas_call(
        functools.partial(_concat_body, split),
        grid=(grid,),
        in_specs=[
            pl.BlockSpec((_BLK, SDR), a_map),
            pl.BlockSpec((_BLK, CDIM), a_map),
            pl.BlockSpec((_BLK, SDR), b_map),
            pl.BlockSpec((_BLK, CDIM), b_map),
        ],
        out_specs=[
            pl.BlockSpec((_BLK, SDR), lambda i: (i, 0)),
            pl.BlockSpec((_BLK, CDIM), lambda i: (i, 0)),
        ],
        out_shape=[
            jax.ShapeDtypeStruct((rows, SDR), jnp.float32),
            jax.ShapeDtypeStruct((rows, CDIM), jnp.float32),
        ],
    )(a_s, a_c, b_s, b_c)


def kernel(sdrs, contents, l1_sdr_bank, l1_content_bank,
           l2_sdr_bank, l2_content_bank, l3_sdr_bank, l3_content_bank):
    sdrs = jax.lax.stop_gradient(sdrs)
    contents = jax.lax.stop_gradient(contents)

    o1s, o1c = _tier_copy(sdrs, contents)
    o2s, o2c = _tier_concat(l1_sdr_bank, l1_content_bank,
                            l2_sdr_bank, l2_content_bank,
                            rows=L2_CAP, a_rows=N, b_row0=N)
    o3s, o3c = _tier_concat(l2_sdr_bank, l2_content_bank,
                            l3_sdr_bank, l3_content_bank,
                            rows=L3_CAP, a_rows=N, b_row0=N)
    return (o1s, o1c, o2s, o2c, o3s, o3c)
